# baseline (speedup 1.0000x reference)
import jax
import jax.numpy as jnp
from jax import lax
from jax.experimental import pallas as pl
from jax.experimental.pallas import tpu as pltpu
from jax.experimental.pallas import tpu_sc as plsc

N = 10000
NP = 10240
E = 320000
D = 64
DW = 16
K = 100
NC = 2
NS = 16
NW = NC * NS
EPW = E // NW
CPW = EPW // K
RPT = NP // NS
RB = 160
LANES = 16
NB = 4


def _zero_vmem(ref, rows, width):
    @pl.loop(0, rows)
    def _(i):
        for k in range(width // LANES):
            ref[i, pl.ds(k * LANES, LANES)] = jnp.zeros((LANES,), jnp.float32)


def _sc_scatter_body(table, row_r, col_r, out, accum, rows_v, cols_v,
                     buf_0, buf_1, buf_2, buf_3, zbuf,
                     gsem_0, gsem_1, gsem_2, gsem_3,
                     ssem_0, ssem_1, ssem_2, ssem_3):
    bufs = (buf_0, buf_1, buf_2, buf_3)
    gsem = (gsem_0, gsem_1, gsem_2, gsem_3)
    ssem = (ssem_0, ssem_1, ssem_2, ssem_3)
    cid = lax.axis_index("c")
    sid = lax.axis_index("s")
    wid = sid * NC + cid

    _zero_vmem(zbuf, RB, D)
    for k in range(RPT // RB):
        pltpu.sync_copy(zbuf, accum.at[pl.ds(sid * RPT + k * RB, RB)])
    pltpu.sync_copy(row_r.at[pl.ds(wid * CPW, CPW)], rows_v)
    pltpu.sync_copy(col_r.at[pl.ds(wid * CPW, CPW)], cols_v)
    plsc.subcore_barrier()

    def fire_g(j, b):
        pltpu.async_copy(table.at[rows_v.at[j]], bufs[b], gsem[b])

    def drain_g(b):
        pltpu.make_async_copy(table.at[pl.ds(0, K)], bufs[b], gsem[b]).wait()

    def fire_s(j, b):
        pltpu.async_copy(bufs[b], accum.at[cols_v.at[j]], ssem[b], add=True)

    def drain_s(b):
        pltpu.make_async_copy(bufs[b], accum.at[pl.ds(0, K)], ssem[b]).wait()

    for b in range(NB):
        fire_g(b, b)

    G = CPW // NB

    @pl.loop(0, G)
    def _(g):
        for b in range(NB):
            drain_g(b)
            fire_s(g * NB + b, b)
        for b in range(NB):
            @pl.when(g < G - 1)
            def _():
                drain_s(b)
                fire_g((g + 1) * NB + b, b)

    for b in range(NB):
        drain_s(b)

    plsc.subcore_barrier()
    for k in range(RPT // RB):
        start = sid * RPT + k * RB
        pltpu.sync_copy(accum.at[pl.ds(start, RB)], zbuf)
        pltpu.sync_copy(zbuf, out.at[cid].at[pl.ds(start, RB)])


def _sc_scatter(table, row_r, col_r):
    mesh = plsc.VectorSubcoreMesh(core_axis_name="c", subcore_axis_name="s")
    return pl.kernel(
        _sc_scatter_body,
        out_type=jax.ShapeDtypeStruct((NC, NP, D), jnp.float32),
        mesh=mesh,
        scratch_types=[
            pltpu.VMEM_SHARED((NP, D), jnp.float32),
            pltpu.VMEM((CPW, K), jnp.int32),
            pltpu.VMEM((CPW, K), jnp.int32),
            pltpu.VMEM((K, D), jnp.float32),
            pltpu.VMEM((K, D), jnp.float32),
            pltpu.VMEM((K, D), jnp.float32),
            pltpu.VMEM((K, D), jnp.float32),
            pltpu.VMEM((RB, D), jnp.float32),
            pltpu.SemaphoreType.DMA,
            pltpu.SemaphoreType.DMA,
            pltpu.SemaphoreType.DMA,
            pltpu.SemaphoreType.DMA,
            pltpu.SemaphoreType.DMA,
            pltpu.SemaphoreType.DMA,
            pltpu.SemaphoreType.DMA,
            pltpu.SemaphoreType.DMA,
        ],
        compiler_params=pltpu.CompilerParams(use_tc_tiling_on_sc=False),
        name="gcn_edge_scatter",
    )(table, row_r, col_r)


def _sc_degree_body(col_r, out, accum, cols_v, ones_v, zbuf, sem):
    cid = lax.axis_index("c")
    sid = lax.axis_index("s")
    wid = sid * NC + cid

    _zero_vmem(zbuf, RB, DW)
    for k in range(RPT // RB):
        pltpu.sync_copy(zbuf, accum.at[pl.ds(sid * RPT + k * RB, RB)])

    @pl.loop(0, K)
    def _(i):
        ones_v[i, pl.ds(0, LANES)] = jnp.ones((LANES,), jnp.float32)

    pltpu.sync_copy(col_r.at[pl.ds(wid * CPW, CPW)], cols_v)
    plsc.subcore_barrier()

    FK = 10

    @pl.loop(0, CPW // FK)
    def _(g):
        for i in range(FK):
            pltpu.async_copy(ones_v, accum.at[cols_v.at[g * FK + i]], sem,
                             add=True)
        for _i in range(FK):
            pltpu.make_async_copy(ones_v, accum.at[pl.ds(0, K)], sem).wait()

    plsc.subcore_barrier()
    for k in range(RPT // RB):
        start = sid * RPT + k * RB
        pltpu.sync_copy(accum.at[pl.ds(start, RB)], zbuf)
        pltpu.sync_copy(zbuf, out.at[cid].at[pl.ds(start, RB)])


def _sc_degree(col_r):
    mesh = plsc.VectorSubcoreMesh(core_axis_name="c", subcore_axis_name="s")
    return pl.kernel(
        _sc_degree_body,
        out_type=jax.ShapeDtypeStruct((NC, NP, DW), jnp.float32),
        mesh=mesh,
        scratch_types=[
            pltpu.VMEM_SHARED((NP, DW), jnp.float32),
            pltpu.VMEM((CPW, K), jnp.int32),
            pltpu.VMEM((K, DW), jnp.float32),
            pltpu.VMEM((RB, DW), jnp.float32),
            pltpu.SemaphoreType.DMA,
        ],
        compiler_params=pltpu.CompilerParams(use_tc_tiling_on_sc=False),
        name="gcn_degree",
    )(col_r)


def _dinv(deg_ref):
    deg = deg_ref[0] + deg_ref[1]
    return lax.rsqrt(deg[:, 0:1] + 1.0)


def _spart(s_ref):
    return s_ref[0] + s_ref[1]


def _tc1_body(deg_ref, x_ref, w_ref, o_ref):
    xw = jnp.dot(x_ref[...], w_ref[...], preferred_element_type=jnp.float32)
    xw = jnp.concatenate([xw, jnp.zeros((NP - N, D), jnp.float32)], axis=0)
    o_ref[...] = xw * _dinv(deg_ref)


def _tc2_body(deg_ref, s_ref, xs_ref, b_ref, w_ref, h_ref, o_ref):
    dinv = _dinv(deg_ref)
    s = _spart(s_ref) + xs_ref[...]
    h = jnp.maximum(s * dinv + b_ref[...], 0.0)
    h_ref[...] = h
    o_ref[...] = jnp.dot(h, w_ref[...],
                         preferred_element_type=jnp.float32) * dinv


def _tc3_body(deg_ref, s_ref, xs_ref, b_ref, h_ref, o_ref):
    dinv = _dinv(deg_ref)
    s = _spart(s_ref) + xs_ref[...]
    y = jnp.maximum(s * dinv + b_ref[...], 0.0)
    o_ref[...] = (y + h_ref[...]) * dinv


def _tc4_body(deg_ref, s_ref, hs_ref, w_ref, b_ref, o_ref):
    dinv = _dinv(deg_ref)
    t = _spart(s_ref) + hs_ref[...]
    z = jnp.dot(t, w_ref[...], preferred_element_type=jnp.float32)
    z = z * dinv + b_ref[...]
    m = jnp.max(z, axis=1, keepdims=True)
    e = z - m
    o_ref[...] = e - jnp.log(jnp.sum(jnp.exp(e), axis=1, keepdims=True))


def _tc(body, out_shape, *args):
    return pl.pallas_call(body, out_shape=out_shape)(*args)


def kernel(x, edge_index, W1, b1, W2, b2, W3, b3):
    f32 = jnp.float32
    row_r = edge_index[0].reshape(NW * CPW, K)
    col_r = edge_index[1].reshape(NW * CPW, K)

    deg_p = _sc_degree(col_r)
    xs1 = _tc(_tc1_body, jax.ShapeDtypeStruct((NP, D), f32), deg_p, x, W1)
    s1p = _sc_scatter(xs1, row_r, col_r)
    h, xs2 = _tc(
        _tc2_body,
        (jax.ShapeDtypeStruct((NP, D), f32),
         jax.ShapeDtypeStruct((NP, D), f32)),
        deg_p, s1p, xs1, b1.reshape(1, D), W2)
    s2p = _sc_scatter(xs2, row_r, col_r)
    hs3 = _tc(_tc3_body, jax.ShapeDtypeStruct((NP, D), f32),
              deg_p, s2p, xs2, b2.reshape(1, D), h)
    s3p = _sc_scatter(hs3, row_r, col_r)
    out = _tc(_tc4_body, jax.ShapeDtypeStruct((NP, 10), f32),
              deg_p, s3p, hs3, W3, b3.reshape(1, 10))
    return out[:N]


# speedup vs baseline: 1.2025x; 1.2025x over previous
import jax
import jax.numpy as jnp
from jax import lax
from jax.experimental import pallas as pl
from jax.experimental.pallas import tpu as pltpu
from jax.experimental.pallas import tpu_sc as plsc

N = 10000
NP = 10240
NH = NP // 2
E = 320000
D = 64
DW = 16
K = 100
NC = 2
NS = 16
NW = NC * NS
EPW = E // NW
CPW = EPW // K
RPT = NP // NS
RB = 160
LANES = 16
NB = 4


def _zero_vmem(ref, rows, width):
    @pl.loop(0, rows)
    def _(i):
        for k in range(width // LANES):
            ref[i, pl.ds(k * LANES, LANES)] = jnp.zeros((LANES,), jnp.float32)


def _sc_scatter_body(table, row_r, col_r, out, accum, rows_v, cols_v,
                     buf_0, buf_1, buf_2, buf_3, zbuf, zwide,
                     gsem_0, gsem_1, gsem_2, gsem_3,
                     ssem_0, ssem_1, ssem_2, ssem_3):
    bufs = (buf_0, buf_1, buf_2, buf_3)
    gsem = (gsem_0, gsem_1, gsem_2, gsem_3)
    ssem = (ssem_0, ssem_1, ssem_2, ssem_3)
    cid = lax.axis_index("c")
    sid = lax.axis_index("s")
    wid = sid * NC + cid

    _zero_vmem(zbuf, RB, D)
    for k in range(RPT // RB):
        pltpu.sync_copy(zbuf, accum.at[pl.ds(sid * RPT + k * RB, RB)])
    pltpu.sync_copy(row_r.at[pl.ds(wid * CPW, CPW)], rows_v)
    pltpu.sync_copy(col_r.at[pl.ds(wid * CPW, CPW)], cols_v)
    plsc.subcore_barrier()

    def fire_g(j, b):
        pltpu.async_copy(table.at[rows_v.at[j]], bufs[b], gsem[b])

    def drain_g(b):
        pltpu.make_async_copy(table.at[pl.ds(0, K)], bufs[b], gsem[b]).wait()

    def fire_s(j, b):
        pltpu.async_copy(bufs[b], accum.at[cols_v.at[j]], ssem[b], add=True)

    def drain_s(b):
        pltpu.make_async_copy(bufs[b], accum.at[pl.ds(0, K)], ssem[b]).wait()

    for b in range(NB):
        fire_g(b, b)

    G = CPW // NB

    @pl.loop(0, G)
    def _(g):
        for b in range(NB):
            drain_g(b)
            fire_s(g * NB + b, b)
        for b in range(NB):
            @pl.when(g < G - 1)
            def _():
                drain_s(b)
                fire_g((g + 1) * NB + b, b)

    for b in range(NB):
        drain_s(b)

    plsc.subcore_barrier()
    for k in range(RPT // RB):
        start = sid * RPT + k * RB
        pltpu.sync_copy(accum.at[pl.ds(start, RB)], zbuf)

        @pl.loop(0, RB // 2)
        def _(p):
            for q in range(D // LANES):
                zwide[p, pl.ds(q * LANES, LANES)] = \
                    zbuf[2 * p, pl.ds(q * LANES, LANES)]
                zwide[p, pl.ds(D + q * LANES, LANES)] = \
                    zbuf[2 * p + 1, pl.ds(q * LANES, LANES)]

        pltpu.sync_copy(zwide, out.at[cid].at[pl.ds(start // 2, RB // 2)])


def _sc_scatter(table, row_r, col_r):
    mesh = plsc.VectorSubcoreMesh(core_axis_name="c", subcore_axis_name="s")
    return pl.kernel(
        _sc_scatter_body,
        out_type=jax.ShapeDtypeStruct((NC, NH, 128), jnp.float32),
        mesh=mesh,
        scratch_types=[
            pltpu.VMEM_SHARED((NP, D), jnp.float32),
            pltpu.VMEM((CPW, K), jnp.int32),
            pltpu.VMEM((CPW, K), jnp.int32),
            pltpu.VMEM((K, D), jnp.float32),
            pltpu.VMEM((K, D), jnp.float32),
            pltpu.VMEM((K, D), jnp.float32),
            pltpu.VMEM((K, D), jnp.float32),
            pltpu.VMEM((RB, D), jnp.float32),
            pltpu.VMEM((RB // 2, 128), jnp.float32),
            pltpu.SemaphoreType.DMA,
            pltpu.SemaphoreType.DMA,
            pltpu.SemaphoreType.DMA,
            pltpu.SemaphoreType.DMA,
            pltpu.SemaphoreType.DMA,
            pltpu.SemaphoreType.DMA,
            pltpu.SemaphoreType.DMA,
            pltpu.SemaphoreType.DMA,
        ],
        compiler_params=pltpu.CompilerParams(use_tc_tiling_on_sc=False),
        name="gcn_edge_scatter",
    )(table, row_r, col_r)


def _sc_degree_body(col_r, out, accum, cols_v, ones_v, zbuf, zwide, sem):
    cid = lax.axis_index("c")
    sid = lax.axis_index("s")
    wid = sid * NC + cid

    _zero_vmem(zbuf, RB, DW)
    for k in range(RPT // RB):
        pltpu.sync_copy(zbuf, accum.at[pl.ds(sid * RPT + k * RB, RB)])

    @pl.loop(0, K)
    def _(i):
        ones_v[i, pl.ds(0, LANES)] = jnp.ones((LANES,), jnp.float32)

    pltpu.sync_copy(col_r.at[pl.ds(wid * CPW, CPW)], cols_v)
    plsc.subcore_barrier()

    FK = 10

    @pl.loop(0, CPW // FK)
    def _(g):
        for i in range(FK):
            pltpu.async_copy(ones_v, accum.at[cols_v.at[g * FK + i]], sem,
                             add=True)
        for _i in range(FK):
            pltpu.make_async_copy(ones_v, accum.at[pl.ds(0, K)], sem).wait()

    plsc.subcore_barrier()
    for k in range(RPT // RB):
        start = sid * RPT + k * RB
        pltpu.sync_copy(accum.at[pl.ds(start, RB)], zbuf)

        @pl.loop(0, RB // 2)
        def _(p):
            va = zbuf[2 * p, pl.ds(0, LANES)]
            vb = zbuf[2 * p + 1, pl.ds(0, LANES)]
            for q in range(D // LANES):
                zwide[p, pl.ds(q * LANES, LANES)] = va
                zwide[p, pl.ds(D + q * LANES, LANES)] = vb

        pltpu.sync_copy(zwide, out.at[cid].at[pl.ds(start // 2, RB // 2)])


def _sc_degree(col_r):
    mesh = plsc.VectorSubcoreMesh(core_axis_name="c", subcore_axis_name="s")
    return pl.kernel(
        _sc_degree_body,
        out_type=jax.ShapeDtypeStruct((NC, NH, 128), jnp.float32),
        mesh=mesh,
        scratch_types=[
            pltpu.VMEM_SHARED((NP, DW), jnp.float32),
            pltpu.VMEM((CPW, K), jnp.int32),
            pltpu.VMEM((K, DW), jnp.float32),
            pltpu.VMEM((RB, DW), jnp.float32),
            pltpu.VMEM((RB // 2, 128), jnp.float32),
            pltpu.SemaphoreType.DMA,
        ],
        compiler_params=pltpu.CompilerParams(use_tc_tiling_on_sc=False),
        name="gcn_degree",
    )(col_r)


def _dinvp(deg_ref):
    return lax.rsqrt(deg_ref[0] + deg_ref[1] + 1.0)


def _tc1_body(deg_ref, xp_ref, w_ref, o_ref):
    xw = jnp.dot(xp_ref[...], w_ref[...], preferred_element_type=jnp.float32)
    xw = jnp.concatenate(
        [xw, jnp.zeros((NH - N // 2, 128), jnp.float32)], axis=0)
    o_ref[...] = xw * _dinvp(deg_ref)


def _tc2_body(deg_ref, s_ref, xs_ref, b_ref, w_ref, h_ref, o_ref):
    dinv = _dinvp(deg_ref)
    s = s_ref[0] + s_ref[1] + xs_ref[...]
    h = jnp.maximum(s * dinv + b_ref[...], 0.0)
    h_ref[...] = h
    o_ref[...] = jnp.dot(h, w_ref[...],
                         preferred_element_type=jnp.float32) * dinv


def _tc3_body(deg_ref, s_ref, xs_ref, b_ref, h_ref, o_ref):
    dinv = _dinvp(deg_ref)
    s = s_ref[0] + s_ref[1] + xs_ref[...]
    y = jnp.maximum(s * dinv + b_ref[...], 0.0)
    o_ref[...] = (y + h_ref[...]) * dinv


def _tc4_body(deg_ref, s_ref, hs_ref, w_ref, b_ref, o_ref):
    dinv = _dinvp(deg_ref)
    t = s_ref[0] + s_ref[1] + hs_ref[...]
    z = jnp.dot(t, w_ref[...], preferred_element_type=jnp.float32)
    dsc = jnp.concatenate([dinv[:, 0:10], dinv[:, D:D + 10]], axis=1)
    z = z * dsc + b_ref[...]

    def lsm(zz):
        m = jnp.max(zz, axis=1, keepdims=True)
        e = zz - m
        return e - jnp.log(jnp.sum(jnp.exp(e), axis=1, keepdims=True))

    o_ref[...] = jnp.concatenate([lsm(z[:, 0:10]), lsm(z[:, 10:20])], axis=1)


def _tc(body, out_shape, *args):
    return pl.pallas_call(body, out_shape=out_shape)(*args)


def _blockdiag(w):
    fi, fo = w.shape
    z = jnp.zeros((fi, fo), w.dtype)
    return jnp.concatenate(
        [jnp.concatenate([w, z], axis=1), jnp.concatenate([z, w], axis=1)],
        axis=0)


def kernel(x, edge_index, W1, b1, W2, b2, W3, b3):
    f32 = jnp.float32
    row_r = edge_index[0].reshape(NW * CPW, K)
    col_r = edge_index[1].reshape(NW * CPW, K)
    xp = x.reshape(N // 2, 256)

    def bp(b):
        return jnp.concatenate([b, b]).reshape(1, -1)

    def tbl(a_pk):
        return jnp.reshape(a_pk, (NP, D))

    deg_p = _sc_degree(col_r)
    xs1 = _tc(_tc1_body, jax.ShapeDtypeStruct((NH, 128), f32),
              deg_p, xp, _blockdiag(W1))
    s1p = _sc_scatter(tbl(xs1), row_r, col_r)
    h, xs2 = _tc(
        _tc2_body,
        (jax.ShapeDtypeStruct((NH, 128), f32),
         jax.ShapeDtypeStruct((NH, 128), f32)),
        deg_p, s1p, xs1, bp(b1), _blockdiag(W2))
    s2p = _sc_scatter(tbl(xs2), row_r, col_r)
    hs3 = _tc(_tc3_body, jax.ShapeDtypeStruct((NH, 128), f32),
              deg_p, s2p, xs2, bp(b2), h)
    s3p = _sc_scatter(tbl(hs3), row_r, col_r)
    out = _tc(_tc4_body, jax.ShapeDtypeStruct((NH, 20), f32),
              deg_p, s3p, hs3, _blockdiag(W3), bp(b3))
    return jnp.reshape(out, (NP, 10))[:N]


# speedup vs baseline: 1.2113x; 1.0073x over previous
import jax
import jax.numpy as jnp
from jax import lax
from jax.experimental import pallas as pl
from jax.experimental.pallas import tpu as pltpu
from jax.experimental.pallas import tpu_sc as plsc

N = 10000
NP = 10240
NH = NP // 2
E = 320000
D = 64
DW = 16
K = 100
NC = 2
NS = 16
NW = NC * NS
EPW = E // NW
CPW = EPW // K
RPT = NP // NS
RB = 160
LANES = 16
NB = 4


def _zero_vmem(ref, rows, width):
    @pl.loop(0, rows)
    def _(i):
        for k in range(width // LANES):
            ref[i, pl.ds(k * LANES, LANES)] = jnp.zeros((LANES,), jnp.float32)


def _sc_scatter_body(table, ei_r, out, accum, rows_v, cols_v,
                     buf_0, buf_1, buf_2, buf_3, zbuf, zwide,
                     gsem_0, gsem_1, gsem_2, gsem_3,
                     ssem_0, ssem_1, ssem_2, ssem_3):
    bufs = (buf_0, buf_1, buf_2, buf_3)
    gsem = (gsem_0, gsem_1, gsem_2, gsem_3)
    ssem = (ssem_0, ssem_1, ssem_2, ssem_3)
    cid = lax.axis_index("c")
    sid = lax.axis_index("s")
    wid = sid * NC + cid

    _zero_vmem(zbuf, RB, D)
    for k in range(RPT // RB):
        pltpu.sync_copy(zbuf, accum.at[pl.ds(sid * RPT + k * RB, RB)])
    pltpu.sync_copy(ei_r.at[0].at[pl.ds(wid * CPW, CPW)], rows_v)
    pltpu.sync_copy(ei_r.at[1].at[pl.ds(wid * CPW, CPW)], cols_v)
    plsc.subcore_barrier()

    def fire_g(j, b):
        pltpu.async_copy(table.at[rows_v.at[j]], bufs[b], gsem[b])

    def drain_g(b):
        pltpu.make_async_copy(table.at[pl.ds(0, K)], bufs[b], gsem[b]).wait()

    def fire_s(j, b):
        pltpu.async_copy(bufs[b], accum.at[cols_v.at[j]], ssem[b], add=True)

    def drain_s(b):
        pltpu.make_async_copy(bufs[b], accum.at[pl.ds(0, K)], ssem[b]).wait()

    for b in range(NB):
        fire_g(b, b)

    G = CPW // NB

    @pl.loop(0, G)
    def _(g):
        for b in range(NB):
            drain_g(b)
            fire_s(g * NB + b, b)
        for b in range(NB):
            @pl.when(g < G - 1)
            def _():
                drain_s(b)
                fire_g((g + 1) * NB + b, b)

    for b in range(NB):
        drain_s(b)

    plsc.subcore_barrier()
    for k in range(RPT // RB):
        start = sid * RPT + k * RB
        pltpu.sync_copy(accum.at[pl.ds(start, RB)], zbuf)

        @pl.loop(0, RB // 2)
        def _(p):
            for q in range(D // LANES):
                zwide[p, pl.ds(q * LANES, LANES)] = \
                    zbuf[2 * p, pl.ds(q * LANES, LANES)]
                zwide[p, pl.ds(D + q * LANES, LANES)] = \
                    zbuf[2 * p + 1, pl.ds(q * LANES, LANES)]

        pltpu.sync_copy(zwide, out.at[cid].at[pl.ds(start // 2, RB // 2)])


def _sc_scatter(table, ei_r):
    mesh = plsc.VectorSubcoreMesh(core_axis_name="c", subcore_axis_name="s")
    return pl.kernel(
        _sc_scatter_body,
        out_type=jax.ShapeDtypeStruct((NC, NH, 128), jnp.float32),
        mesh=mesh,
        scratch_types=[
            pltpu.VMEM_SHARED((NP, D), jnp.float32),
            pltpu.VMEM((CPW, K), jnp.int32),
            pltpu.VMEM((CPW, K), jnp.int32),
            pltpu.VMEM((K, D), jnp.float32),
            pltpu.VMEM((K, D), jnp.float32),
            pltpu.VMEM((K, D), jnp.float32),
            pltpu.VMEM((K, D), jnp.float32),
            pltpu.VMEM((RB, D), jnp.float32),
            pltpu.VMEM((RB // 2, 128), jnp.float32),
            pltpu.SemaphoreType.DMA,
            pltpu.SemaphoreType.DMA,
            pltpu.SemaphoreType.DMA,
            pltpu.SemaphoreType.DMA,
            pltpu.SemaphoreType.DMA,
            pltpu.SemaphoreType.DMA,
            pltpu.SemaphoreType.DMA,
            pltpu.SemaphoreType.DMA,
        ],
        compiler_params=pltpu.CompilerParams(use_tc_tiling_on_sc=False),
        name="gcn_edge_scatter",
    )(table, ei_r)


def _sc_degree_body(ei_r, out, accum, cols_v, ones_v, zbuf, zwide, sem):
    cid = lax.axis_index("c")
    sid = lax.axis_index("s")
    wid = sid * NC + cid

    _zero_vmem(zbuf, RB, DW)
    for k in range(RPT // RB):
        pltpu.sync_copy(zbuf, accum.at[pl.ds(sid * RPT + k * RB, RB)])

    @pl.loop(0, K)
    def _(i):
        ones_v[i, pl.ds(0, LANES)] = jnp.ones((LANES,), jnp.float32)

    pltpu.sync_copy(ei_r.at[1].at[pl.ds(wid * CPW, CPW)], cols_v)
    plsc.subcore_barrier()

    FK = 10

    @pl.loop(0, CPW // FK)
    def _(g):
        for i in range(FK):
            pltpu.async_copy(ones_v, accum.at[cols_v.at[g * FK + i]], sem,
                             add=True)
        for _i in range(FK):
            pltpu.make_async_copy(ones_v, accum.at[pl.ds(0, K)], sem).wait()

    plsc.subcore_barrier()
    for k in range(RPT // RB):
        start = sid * RPT + k * RB
        pltpu.sync_copy(accum.at[pl.ds(start, RB)], zbuf)

        @pl.loop(0, RB // 2)
        def _(p):
            va = zbuf[2 * p, pl.ds(0, LANES)]
            vb = zbuf[2 * p + 1, pl.ds(0, LANES)]
            for q in range(D // LANES):
                zwide[p, pl.ds(q * LANES, LANES)] = va
                zwide[p, pl.ds(D + q * LANES, LANES)] = vb

        pltpu.sync_copy(zwide, out.at[cid].at[pl.ds(start // 2, RB // 2)])


def _sc_degree(ei_r):
    mesh = plsc.VectorSubcoreMesh(core_axis_name="c", subcore_axis_name="s")
    return pl.kernel(
        _sc_degree_body,
        out_type=jax.ShapeDtypeStruct((NC, NH, 128), jnp.float32),
        mesh=mesh,
        scratch_types=[
            pltpu.VMEM_SHARED((NP, DW), jnp.float32),
            pltpu.VMEM((CPW, K), jnp.int32),
            pltpu.VMEM((K, DW), jnp.float32),
            pltpu.VMEM((RB, DW), jnp.float32),
            pltpu.VMEM((RB // 2, 128), jnp.float32),
            pltpu.SemaphoreType.DMA,
        ],
        compiler_params=pltpu.CompilerParams(use_tc_tiling_on_sc=False),
        name="gcn_degree",
    )(ei_r)


def _dinvp(deg_ref):
    return lax.rsqrt(deg_ref[0] + deg_ref[1] + 1.0)


def _tc1_body(deg_ref, xp_ref, w_ref, o_ref):
    xw = jnp.dot(xp_ref[...], w_ref[...], preferred_element_type=jnp.float32)
    xw = jnp.concatenate(
        [xw, jnp.zeros((NH - N // 2, 128), jnp.float32)], axis=0)
    o_ref[...] = xw * _dinvp(deg_ref)


def _tc2_body(deg_ref, s_ref, xs_ref, b_ref, w_ref, h_ref, o_ref):
    dinv = _dinvp(deg_ref)
    s = s_ref[0] + s_ref[1] + xs_ref[...]
    h = jnp.maximum(s * dinv + b_ref[...], 0.0)
    h_ref[...] = h
    o_ref[...] = jnp.dot(h, w_ref[...],
                         preferred_element_type=jnp.float32) * dinv


def _tc3_body(deg_ref, s_ref, xs_ref, b_ref, h_ref, o_ref):
    dinv = _dinvp(deg_ref)
    s = s_ref[0] + s_ref[1] + xs_ref[...]
    y = jnp.maximum(s * dinv + b_ref[...], 0.0)
    o_ref[...] = (y + h_ref[...]) * dinv


def _tc4_body(deg_ref, s_ref, hs_ref, w_ref, b_ref, o_ref):
    dinv = _dinvp(deg_ref)
    t = s_ref[0] + s_ref[1] + hs_ref[...]
    z = jnp.dot(t, w_ref[...], preferred_element_type=jnp.float32)
    dsc = jnp.concatenate([dinv[:, 0:10], dinv[:, D:D + 10]], axis=1)
    z = z * dsc + b_ref[...]

    def lsm(zz):
        m = jnp.max(zz, axis=1, keepdims=True)
        e = zz - m
        return e - jnp.log(jnp.sum(jnp.exp(e), axis=1, keepdims=True))

    o_ref[...] = jnp.concatenate([lsm(z[:, 0:10]), lsm(z[:, 10:20])], axis=1)


def _tc(body, out_shape, *args):
    return pl.pallas_call(body, out_shape=out_shape)(*args)


def _blockdiag(w):
    fi, fo = w.shape
    z = jnp.zeros((fi, fo), w.dtype)
    return jnp.concatenate(
        [jnp.concatenate([w, z], axis=1), jnp.concatenate([z, w], axis=1)],
        axis=0)


def kernel(x, edge_index, W1, b1, W2, b2, W3, b3):
    f32 = jnp.float32
    ei_r = edge_index.reshape(2, NW * CPW, K)
    xp = x.reshape(N // 2, 256)

    def bp(b):
        return jnp.concatenate([b, b]).reshape(1, -1)

    def tbl(a_pk):
        return jnp.reshape(a_pk, (NP, D))

    deg_p = _sc_degree(ei_r)
    xs1 = _tc(_tc1_body, jax.ShapeDtypeStruct((NH, 128), f32),
              deg_p, xp, _blockdiag(W1))
    s1p = _sc_scatter(tbl(xs1), ei_r)
    h, xs2 = _tc(
        _tc2_body,
        (jax.ShapeDtypeStruct((NH, 128), f32),
         jax.ShapeDtypeStruct((NH, 128), f32)),
        deg_p, s1p, xs1, bp(b1), _blockdiag(W2))
    s2p = _sc_scatter(tbl(xs2), ei_r)
    hs3 = _tc(_tc3_body, jax.ShapeDtypeStruct((NH, 128), f32),
              deg_p, s2p, xs2, bp(b2), h)
    s3p = _sc_scatter(tbl(hs3), ei_r)
    out = _tc(_tc4_body, jax.ShapeDtypeStruct((NH, 20), f32),
              deg_p, s3p, hs3, _blockdiag(W3), bp(b3))
    return jnp.reshape(out, (NP, 10))[:N]


# speedup vs baseline: 1.2514x; 1.0331x over previous
import jax
import jax.numpy as jnp
from jax import lax
from jax.experimental import pallas as pl
from jax.experimental.pallas import tpu as pltpu
from jax.experimental.pallas import tpu_sc as plsc

N = 10000
NP = 10240
NH = NP // 2
E = 320000
D = 64
DW = 16
K = 125
NC = 2
NS = 16
NW = NC * NS
EPW = E // NW
CPW = EPW // K
RPT = NP // NS
RB = 160
LANES = 16
NB = 4


def _zero_vmem(ref, rows, width):
    @pl.loop(0, rows)
    def _(i):
        for k in range(width // LANES):
            ref[i, pl.ds(k * LANES, LANES)] = jnp.zeros((LANES,), jnp.float32)


def _sc_scatter_body(table, ei_r, out, accum, rows_v, cols_v,
                     buf_0, buf_1, buf_2, buf_3, zbuf, zwide,
                     gsem_0, gsem_1, gsem_2, gsem_3,
                     ssem_0, ssem_1, ssem_2, ssem_3):
    bufs = (buf_0, buf_1, buf_2, buf_3)
    gsem = (gsem_0, gsem_1, gsem_2, gsem_3)
    ssem = (ssem_0, ssem_1, ssem_2, ssem_3)
    cid = lax.axis_index("c")
    sid = lax.axis_index("s")
    wid = sid * NC + cid

    _zero_vmem(zbuf, RB, D)
    for k in range(RPT // RB):
        pltpu.sync_copy(zbuf, accum.at[pl.ds(sid * RPT + k * RB, RB)])
    pltpu.sync_copy(ei_r.at[0].at[pl.ds(wid * CPW, CPW)], rows_v)
    pltpu.sync_copy(ei_r.at[1].at[pl.ds(wid * CPW, CPW)], cols_v)
    plsc.subcore_barrier()

    def fire_g(j, b):
        pltpu.async_copy(table.at[rows_v.at[j]], bufs[b], gsem[b])

    def drain_g(b):
        pltpu.make_async_copy(table.at[pl.ds(0, K)], bufs[b], gsem[b]).wait()

    def fire_s(j, b):
        pltpu.async_copy(bufs[b], accum.at[cols_v.at[j]], ssem[b], add=True)

    def drain_s(b):
        pltpu.make_async_copy(bufs[b], accum.at[pl.ds(0, K)], ssem[b]).wait()

    for b in range(NB):
        fire_g(b, b)

    G = CPW // NB

    @pl.loop(0, G)
    def _(g):
        for b in range(NB):
            drain_g(b)
            fire_s(g * NB + b, b)
        for b in range(NB):
            @pl.when(g < G - 1)
            def _():
                drain_s(b)
                fire_g((g + 1) * NB + b, b)

    for b in range(NB):
        drain_s(b)

    plsc.subcore_barrier()
    for k in range(RPT // RB):
        start = sid * RPT + k * RB
        pltpu.sync_copy(accum.at[pl.ds(start, RB)], zbuf)

        @pl.loop(0, RB // 2)
        def _(p):
            for q in range(D // LANES):
                zwide[p, pl.ds(q * LANES, LANES)] = \
                    zbuf[2 * p, pl.ds(q * LANES, LANES)]
                zwide[p, pl.ds(D + q * LANES, LANES)] = \
                    zbuf[2 * p + 1, pl.ds(q * LANES, LANES)]

        pltpu.sync_copy(zwide, out.at[cid].at[pl.ds(start // 2, RB // 2)])


def _sc_scatter(table, ei_r):
    mesh = plsc.VectorSubcoreMesh(core_axis_name="c", subcore_axis_name="s")
    return pl.kernel(
        _sc_scatter_body,
        out_type=jax.ShapeDtypeStruct((NC, NH, 128), jnp.float32),
        mesh=mesh,
        scratch_types=[
            pltpu.VMEM_SHARED((NP, D), jnp.float32),
            pltpu.VMEM((CPW, K), jnp.int32),
            pltpu.VMEM((CPW, K), jnp.int32),
            pltpu.VMEM((K, D), jnp.float32),
            pltpu.VMEM((K, D), jnp.float32),
            pltpu.VMEM((K, D), jnp.float32),
            pltpu.VMEM((K, D), jnp.float32),
            pltpu.VMEM((RB, D), jnp.float32),
            pltpu.VMEM((RB // 2, 128), jnp.float32),
            pltpu.SemaphoreType.DMA,
            pltpu.SemaphoreType.DMA,
            pltpu.SemaphoreType.DMA,
            pltpu.SemaphoreType.DMA,
            pltpu.SemaphoreType.DMA,
            pltpu.SemaphoreType.DMA,
            pltpu.SemaphoreType.DMA,
            pltpu.SemaphoreType.DMA,
        ],
        compiler_params=pltpu.CompilerParams(use_tc_tiling_on_sc=False),
        name="gcn_edge_scatter",
    )(table, ei_r)


def _sc_degree_body(ei_r, out, accum, cols_v, ones_v, zbuf, zwide, sem):
    cid = lax.axis_index("c")
    sid = lax.axis_index("s")
    wid = sid * NC + cid

    _zero_vmem(zbuf, RB, DW)
    for k in range(RPT // RB):
        pltpu.sync_copy(zbuf, accum.at[pl.ds(sid * RPT + k * RB, RB)])

    @pl.loop(0, K)
    def _(i):
        ones_v[i, pl.ds(0, LANES)] = jnp.ones((LANES,), jnp.float32)

    pltpu.sync_copy(ei_r.at[1].at[pl.ds(wid * CPW, CPW)], cols_v)
    plsc.subcore_barrier()

    FK = 10

    @pl.loop(0, CPW // FK)
    def _(g):
        for i in range(FK):
            pltpu.async_copy(ones_v, accum.at[cols_v.at[g * FK + i]], sem,
                             add=True)
        for _i in range(FK):
            pltpu.make_async_copy(ones_v, accum.at[pl.ds(0, K)], sem).wait()

    plsc.subcore_barrier()
    for k in range(RPT // RB):
        start = sid * RPT + k * RB
        pltpu.sync_copy(accum.at[pl.ds(start, RB)], zbuf)

        @pl.loop(0, RB // 2)
        def _(p):
            va = zbuf[2 * p, pl.ds(0, LANES)]
            vb = zbuf[2 * p + 1, pl.ds(0, LANES)]
            for q in range(D // LANES):
                zwide[p, pl.ds(q * LANES, LANES)] = va
                zwide[p, pl.ds(D + q * LANES, LANES)] = vb

        pltpu.sync_copy(zwide, out.at[cid].at[pl.ds(start // 2, RB // 2)])


def _sc_degree(ei_r):
    mesh = plsc.VectorSubcoreMesh(core_axis_name="c", subcore_axis_name="s")
    return pl.kernel(
        _sc_degree_body,
        out_type=jax.ShapeDtypeStruct((NC, NH, 128), jnp.float32),
        mesh=mesh,
        scratch_types=[
            pltpu.VMEM_SHARED((NP, DW), jnp.float32),
            pltpu.VMEM((CPW, K), jnp.int32),
            pltpu.VMEM((K, DW), jnp.float32),
            pltpu.VMEM((RB, DW), jnp.float32),
            pltpu.VMEM((RB // 2, 128), jnp.float32),
            pltpu.SemaphoreType.DMA,
        ],
        compiler_params=pltpu.CompilerParams(use_tc_tiling_on_sc=False),
        name="gcn_degree",
    )(ei_r)


def _dinvp(deg_ref):
    return lax.rsqrt(deg_ref[0] + deg_ref[1] + 1.0)


def _tc1_body(deg_ref, xp_ref, w_ref, o_ref):
    xw = jnp.dot(xp_ref[...], w_ref[...], preferred_element_type=jnp.float32)
    xw = jnp.concatenate(
        [xw, jnp.zeros((NH - N // 2, 128), jnp.float32)], axis=0)
    o_ref[...] = xw * _dinvp(deg_ref)


def _tc2_body(deg_ref, s_ref, xs_ref, b_ref, w_ref, h_ref, o_ref):
    dinv = _dinvp(deg_ref)
    s = s_ref[0] + s_ref[1] + xs_ref[...]
    h = jnp.maximum(s * dinv + b_ref[...], 0.0)
    h_ref[...] = h
    o_ref[...] = jnp.dot(h, w_ref[...],
                         preferred_element_type=jnp.float32) * dinv


def _tc3_body(deg_ref, s_ref, xs_ref, b_ref, h_ref, o_ref):
    dinv = _dinvp(deg_ref)
    s = s_ref[0] + s_ref[1] + xs_ref[...]
    y = jnp.maximum(s * dinv + b_ref[...], 0.0)
    o_ref[...] = (y + h_ref[...]) * dinv


def _tc4_body(deg_ref, s_ref, hs_ref, w_ref, b_ref, o_ref):
    dinv = _dinvp(deg_ref)
    t = s_ref[0] + s_ref[1] + hs_ref[...]
    z = jnp.dot(t, w_ref[...], preferred_element_type=jnp.float32)
    dsc = jnp.concatenate([dinv[:, 0:10], dinv[:, D:D + 10]], axis=1)
    z = z * dsc + b_ref[...]

    def lsm(zz):
        m = jnp.max(zz, axis=1, keepdims=True)
        e = zz - m
        return e - jnp.log(jnp.sum(jnp.exp(e), axis=1, keepdims=True))

    o_ref[...] = jnp.concatenate([lsm(z[:, 0:10]), lsm(z[:, 10:20])], axis=1)


def _tc(body, out_shape, *args):
    return pl.pallas_call(body, out_shape=out_shape)(*args)


def _blockdiag(w):
    fi, fo = w.shape
    z = jnp.zeros((fi, fo), w.dtype)
    return jnp.concatenate(
        [jnp.concatenate([w, z], axis=1), jnp.concatenate([z, w], axis=1)],
        axis=0)


def kernel(x, edge_index, W1, b1, W2, b2, W3, b3):
    f32 = jnp.float32
    ei_r = edge_index.reshape(2, NW * CPW, K)
    xp = x.reshape(N // 2, 256)

    def bp(b):
        return jnp.concatenate([b, b]).reshape(1, -1)

    def tbl(a_pk):
        return jnp.reshape(a_pk, (NP, D))

    deg_p = _sc_degree(ei_r)
    xs1 = _tc(_tc1_body, jax.ShapeDtypeStruct((NH, 128), f32),
              deg_p, xp, _blockdiag(W1))
    s1p = _sc_scatter(tbl(xs1), ei_r)
    h, xs2 = _tc(
        _tc2_body,
        (jax.ShapeDtypeStruct((NH, 128), f32),
         jax.ShapeDtypeStruct((NH, 128), f32)),
        deg_p, s1p, xs1, bp(b1), _blockdiag(W2))
    s2p = _sc_scatter(tbl(xs2), ei_r)
    hs3 = _tc(_tc3_body, jax.ShapeDtypeStruct((NH, 128), f32),
              deg_p, s2p, xs2, bp(b2), h)
    s3p = _sc_scatter(tbl(hs3), ei_r)
    out = _tc(_tc4_body, jax.ShapeDtypeStruct((NH, 20), f32),
              deg_p, s3p, hs3, _blockdiag(W3), bp(b3))
    return jnp.reshape(out, (NP, 10))[:N]
